# baseline (speedup 1.0000x reference)
import functools

import jax
import jax.numpy as jnp
from jax import lax
from jax.experimental import pallas as pl
from jax.experimental.pallas import tpu as pltpu
from jax.experimental.pallas import tpu_sc as plsc

_V = 8192
_D = 256
_B = 16384
_NC = 2
_NS = 16
_CHUNK = 128
_NBUF = 3

_SIZES0 = (128, 128, 128, 64)
_SIZES1 = (128, 128, 128, 128, 64)
_ROWS0 = sum(_SIZES0)
_ROWS1 = sum(_SIZES1)
assert _NS * (_ROWS0 + _ROWS1) == _B

_mesh = plsc.VectorSubcoreMesh(core_axis_name="c", subcore_axis_name="s")


def _pipeline(idx_hbm, table_hbm, out_hbm, idx_v, bufs, gsems, wsems,
              base, sizes):
    n = len(sizes)
    rows = sum(sizes)
    offs = [sum(sizes[:i]) for i in range(n)]
    pltpu.sync_copy(idx_hbm.at[pl.ds(base, rows)], idx_v.at[pl.ds(0, rows)])
    gcp = [None] * n
    wcp = [None] * n
    for c in range(n):
        b = c % _NBUF
        if c >= _NBUF:
            wcp[c - _NBUF].wait()
        gcp[c] = pltpu.async_copy(
            table_hbm.at[idx_v.at[pl.ds(offs[c], sizes[c])]],
            bufs[b].at[pl.ds(0, sizes[c])],
            gsems[b],
        )
        if c >= 1:
            p = c - 1
            gcp[p].wait()
            wcp[p] = pltpu.async_copy(
                bufs[p % _NBUF].at[pl.ds(0, sizes[p])],
                out_hbm.at[pl.ds(base + offs[p], sizes[p])],
                wsems[p % _NBUF],
            )
    last = n - 1
    gcp[last].wait()
    wcp[last] = pltpu.async_copy(
        bufs[last % _NBUF].at[pl.ds(0, sizes[last])],
        out_hbm.at[pl.ds(base + offs[last], sizes[last])],
        wsems[last % _NBUF],
    )
    for c in range(max(0, n - _NBUF), n):
        wcp[c].wait()


@functools.partial(
    pl.kernel,
    mesh=_mesh,
    out_type=jax.ShapeDtypeStruct((_B, _D), jnp.float32),
    scratch_types=[
        pltpu.VMEM((max(_ROWS0, _ROWS1),), jnp.int32),
    ]
    + [pltpu.VMEM((_CHUNK, _D), jnp.float32) for _ in range(_NBUF)]
    + [pltpu.SemaphoreType.DMA for _ in range(2 * _NBUF)],
)
def _sc_gather(idx_hbm, table_hbm, out_hbm, idx_v, *scratch):
    bufs = scratch[:_NBUF]
    gsems = scratch[_NBUF:2 * _NBUF]
    wsems = scratch[2 * _NBUF:3 * _NBUF]
    cid = lax.axis_index("c")
    sid = lax.axis_index("s")

    @pl.when(cid == 0)
    def _():
        _pipeline(idx_hbm, table_hbm, out_hbm, idx_v, bufs, gsems, wsems,
                  sid * _ROWS0, _SIZES0)

    @pl.when(cid == 1)
    def _():
        _pipeline(idx_hbm, table_hbm, out_hbm, idx_v, bufs, gsems, wsems,
                  _NS * _ROWS0 + sid * _ROWS1, _SIZES1)


def kernel(cls_ids, vec):
    return _sc_gather(cls_ids, vec)


# speedup vs baseline: 1.0151x; 1.0151x over previous
import functools

import jax
import jax.numpy as jnp
from jax import lax
from jax.experimental import pallas as pl
from jax.experimental.pallas import tpu as pltpu
from jax.experimental.pallas import tpu_sc as plsc

_V = 8192
_D = 256
_B = 16384
_NC = 2
_NS = 16
_NW = _NC * _NS
_BPW = _B // _NW
_CHUNK = 128
_NCHUNK = _BPW // _CHUNK
_NBUF = 3

_mesh = plsc.VectorSubcoreMesh(core_axis_name="c", subcore_axis_name="s")


@functools.partial(
    pl.kernel,
    mesh=_mesh,
    out_type=jax.ShapeDtypeStruct((_B, _D), jnp.float32),
    scratch_types=[
        pltpu.VMEM((_NCHUNK, _CHUNK), jnp.int32),
    ]
    + [pltpu.VMEM((_CHUNK, _D), jnp.float32) for _ in range(_NBUF)]
    + [pltpu.SemaphoreType.DMA for _ in range(2 * _NBUF)],
)
def _sc_gather(idx_hbm, table_hbm, out_hbm, idx_v, *scratch):
    bufs = scratch[:_NBUF]
    gsems = scratch[_NBUF:2 * _NBUF]
    wsems = scratch[2 * _NBUF:]
    wid = lax.axis_index("s") * _NC + lax.axis_index("c")
    base = wid * _BPW
    pltpu.sync_copy(idx_hbm.at[wid], idx_v)
    gcp = [None] * _NCHUNK
    wcp = [None] * _NCHUNK
    for c in range(_NCHUNK):
        b = c % _NBUF
        if c >= _NBUF:
            wcp[c - _NBUF].wait()
        gcp[c] = pltpu.async_copy(table_hbm.at[idx_v.at[c]], bufs[b], gsems[b])
        if c >= 1:
            p = c - 1
            gcp[p].wait()
            wcp[p] = pltpu.async_copy(
                bufs[p % _NBUF],
                out_hbm.at[pl.ds(base + p * _CHUNK, _CHUNK)],
                wsems[p % _NBUF],
            )
    last = _NCHUNK - 1
    gcp[last].wait()
    wcp[last] = pltpu.async_copy(
        bufs[last % _NBUF],
        out_hbm.at[pl.ds(base + last * _CHUNK, _CHUNK)],
        wsems[last % _NBUF],
    )
    for c in range(max(0, _NCHUNK - _NBUF), _NCHUNK):
        wcp[c].wait()


def kernel(cls_ids, vec):
    idx3 = cls_ids.reshape(_NW, _NCHUNK, _CHUNK)
    return _sc_gather(idx3, vec)


# speedup vs baseline: 1.0157x; 1.0006x over previous
import functools

import jax
import jax.numpy as jnp
from jax import lax
from jax.experimental import pallas as pl
from jax.experimental.pallas import tpu as pltpu
from jax.experimental.pallas import tpu_sc as plsc

_V = 8192
_D = 256
_B = 16384
_NC = 2
_NS = 16
_NW = _NC * _NS
_BPW = _B // _NW
_CHUNK = 128
_NCHUNK = _BPW // _CHUNK
_NBUF = 3

_mesh = plsc.VectorSubcoreMesh(core_axis_name="c", subcore_axis_name="s")


@functools.partial(
    pl.kernel,
    mesh=_mesh,
    out_type=jax.ShapeDtypeStruct((_B, _D), jnp.float32),
    scratch_types=[
        pltpu.VMEM((_NCHUNK, _CHUNK), jnp.int32),
    ]
    + [pltpu.VMEM((_CHUNK, _D), jnp.float32) for _ in range(_NBUF)]
    + [pltpu.SemaphoreType.DMA for _ in range(2 * _NBUF + 2)],
)
def _sc_gather(idx_hbm, table_hbm, out_hbm, idx_v, *scratch):
    bufs = scratch[:_NBUF]
    gsems = scratch[_NBUF:2 * _NBUF]
    wsems = scratch[2 * _NBUF:3 * _NBUF]
    isem0, isem1 = scratch[3 * _NBUF:]
    wid = lax.axis_index("s") * _NC + lax.axis_index("c")
    base = wid * _BPW
    icp0 = pltpu.async_copy(idx_hbm.at[wid].at[pl.ds(0, 1)],
                            idx_v.at[pl.ds(0, 1)], isem0)
    icp1 = pltpu.async_copy(idx_hbm.at[wid].at[pl.ds(1, _NCHUNK - 1)],
                            idx_v.at[pl.ds(1, _NCHUNK - 1)], isem1)
    gcp = [None] * _NCHUNK
    wcp = [None] * _NCHUNK
    for c in range(_NCHUNK):
        b = c % _NBUF
        if c >= _NBUF:
            wcp[c - _NBUF].wait()
        if c == 0:
            icp0.wait()
        elif c == 1:
            icp1.wait()
        gcp[c] = pltpu.async_copy(table_hbm.at[idx_v.at[c]], bufs[b], gsems[b])
        if c >= 1:
            p = c - 1
            gcp[p].wait()
            wcp[p] = pltpu.async_copy(
                bufs[p % _NBUF],
                out_hbm.at[pl.ds(base + p * _CHUNK, _CHUNK)],
                wsems[p % _NBUF],
            )
    last = _NCHUNK - 1
    gcp[last].wait()
    wcp[last] = pltpu.async_copy(
        bufs[last % _NBUF],
        out_hbm.at[pl.ds(base + last * _CHUNK, _CHUNK)],
        wsems[last % _NBUF],
    )
    for c in range(max(0, _NCHUNK - _NBUF), _NCHUNK):
        wcp[c].wait()


def kernel(cls_ids, vec):
    idx3 = cls_ids.reshape(_NW, _NCHUNK, _CHUNK)
    return _sc_gather(idx3, vec)
